# baseline (speedup 1.0000x reference)
import functools

import numpy as np
import jax
import jax.numpy as jnp
from jax import lax
from jax.experimental import pallas as pl
from jax.experimental.pallas import tpu as pltpu
from jax.experimental.pallas import tpu_sc as plsc

_NC = 2
_NS = 16
_NW = _NC * _NS
_CH = 128
_NBUF = 4


def _col_perm(D):
    perm = np.empty(D, np.int32)
    for g in range(D // 32):
        for k in range(16):
            perm[32 * g + 2 * k] = 32 * g + k
            perm[32 * g + 2 * k + 1] = 32 * g + 16 + k
    return perm


@functools.lru_cache(maxsize=None)
def _make_gather(N, D):
    b_per_w = N // _NW
    n_ch = b_per_w // _CH
    n_outer = n_ch // _NBUF
    assert N == _NW * n_outer * _NBUF * _CH and n_outer >= 2
    W = D // 2
    ngrp = W // 16

    mesh = plsc.VectorSubcoreMesh(core_axis_name="c", subcore_axis_name="s")

    @functools.partial(
        pl.kernel,
        out_type=jax.ShapeDtypeStruct((N, D), jnp.int32),
        mesh=mesh,
        compiler_params=pltpu.CompilerParams(use_tc_tiling_on_sc=False),
        scratch_types=[
            pltpu.VMEM((n_ch, _CH), jnp.int32),
            pltpu.VMEM((_NBUF, _CH, W), jnp.int32),
            pltpu.VMEM((_NBUF, _CH, D), jnp.int32),
            pltpu.SemaphoreType.DMA,
        ]
        + [pltpu.SemaphoreType.DMA for _ in range(_NBUF)]
        + [pltpu.SemaphoreType.DMA for _ in range(_NBUF)],
    )
    def gather_kernel(tab_hbm, idx_hbm, out_hbm, idx_v, pk_v, rows_v, isem, *sems):
        gsem = sems[:_NBUF]
        ssem = sems[_NBUF:]
        wid = lax.axis_index("s") * _NC + lax.axis_index("c")
        base = wid * b_per_w

        pltpu.async_copy(idx_hbm.at[wid], idx_v, isem).wait()

        hi_mask = jnp.full((16,), -65536, jnp.int32)

        def widen(b):
            def row_body(r, carry):
                pk_row = pk_v.at[b, r]
                out_row = rows_v.at[b, r]
                for g in range(ngrp):
                    x = pk_row[pl.ds(16 * g, 16)]
                    out_row[pl.ds(32 * g, 16)] = lax.shift_left(x, 16)
                    out_row[pl.ds(32 * g + 16, 16)] = lax.bitwise_and(x, hi_mask)
                return carry

            lax.fori_loop(0, _CH, row_body, 0)

        for b in range(_NBUF):
            pltpu.async_copy(tab_hbm.at[idx_v.at[b]], pk_v.at[b], gsem[b])

        for b in range(_NBUF):
            pltpu.make_async_copy(
                tab_hbm.at[idx_v.at[b]], pk_v.at[b], gsem[b]
            ).wait()
            widen(b)
            pltpu.async_copy(
                rows_v.at[b], out_hbm.at[pl.ds(base + b * _CH, _CH)], ssem[b]
            )
            pltpu.async_copy(
                tab_hbm.at[idx_v.at[b + _NBUF]], pk_v.at[b], gsem[b]
            )

        def outer(jj, carry):
            for b in range(_NBUF):
                j = jj * _NBUF + b
                pltpu.make_async_copy(
                    tab_hbm.at[idx_v.at[j]], pk_v.at[b], gsem[b]
                ).wait()
                pltpu.make_async_copy(
                    rows_v.at[b],
                    out_hbm.at[pl.ds(base + (j - _NBUF) * _CH, _CH)],
                    ssem[b],
                ).wait()
                widen(b)
                pltpu.async_copy(
                    rows_v.at[b], out_hbm.at[pl.ds(base + j * _CH, _CH)], ssem[b]
                )
                pltpu.async_copy(
                    tab_hbm.at[idx_v.at[j + _NBUF]], pk_v.at[b], gsem[b]
                )
            return carry

        lax.fori_loop(1, n_outer - 1, outer, 0)

        for b in range(_NBUF):
            j = (n_outer - 1) * _NBUF + b
            pltpu.make_async_copy(
                tab_hbm.at[idx_v.at[j]], pk_v.at[b], gsem[b]
            ).wait()
            pltpu.make_async_copy(
                rows_v.at[b],
                out_hbm.at[pl.ds(base + (j - _NBUF) * _CH, _CH)],
                ssem[b],
            ).wait()
            widen(b)
            pltpu.async_copy(
                rows_v.at[b], out_hbm.at[pl.ds(base + j * _CH, _CH)], ssem[b]
            )
        for b in range(_NBUF):
            j = (n_outer - 1) * _NBUF + b
            pltpu.make_async_copy(
                rows_v.at[b], out_hbm.at[pl.ds(base + j * _CH, _CH)], ssem[b]
            ).wait()

    return gather_kernel


def kernel(input, table):
    B, H = input.shape
    V, D = table.shape
    N = B * H
    b_per_w = N // _NW
    tp = table[:, _col_perm(D)].astype(jnp.bfloat16)
    t32 = jax.lax.bitcast_convert_type(tp.reshape(V, D // 2, 2), jnp.int32)
    idx = input.reshape(_NW, b_per_w // _CH, _CH).astype(jnp.int32)
    out = _make_gather(N, D)(t32, idx)
    return jax.lax.bitcast_convert_type(out, jnp.float32).reshape(B, H, D)


# speedup vs baseline: 1.0026x; 1.0026x over previous
import functools

import numpy as np
import jax
import jax.numpy as jnp
from jax import lax
from jax.experimental import pallas as pl
from jax.experimental.pallas import tpu as pltpu
from jax.experimental.pallas import tpu_sc as plsc

_NC = 2
_NS = 16
_NW = _NC * _NS
_CH = 128
_NBUF = 4


def _col_perm(D):
    perm = np.empty(D, np.int32)
    for g in range(D // 32):
        for k in range(16):
            perm[32 * g + 2 * k] = 32 * g + k
            perm[32 * g + 2 * k + 1] = 32 * g + 16 + k
    return perm


@functools.lru_cache(maxsize=None)
def _make_gather(N, D):
    b_per_w = N // _NW
    n_ch = b_per_w // _CH
    n_outer = n_ch // _NBUF
    assert N == _NW * n_outer * _NBUF * _CH and n_outer >= 2
    W = D // 2
    ngrp = W // 16

    mesh = plsc.VectorSubcoreMesh(core_axis_name="c", subcore_axis_name="s")

    @functools.partial(
        pl.kernel,
        out_type=jax.ShapeDtypeStruct((N, D), jnp.int32),
        mesh=mesh,
        compiler_params=pltpu.CompilerParams(use_tc_tiling_on_sc=False),
        scratch_types=[
            pltpu.VMEM((n_ch, _CH), jnp.int32),
            pltpu.VMEM((_NBUF, _CH, W), jnp.int32),
            pltpu.VMEM((_NBUF, _CH, D), jnp.int32),
            pltpu.SemaphoreType.DMA,
        ]
        + [pltpu.SemaphoreType.DMA for _ in range(_NBUF)]
        + [pltpu.SemaphoreType.DMA for _ in range(_NBUF)],
    )
    def gather_kernel(tab_hbm, idx_hbm, out_hbm, idx_v, pk_v, rows_v, isem, *sems):
        gsem = sems[:_NBUF]
        ssem = sems[_NBUF:]
        wid = lax.axis_index("s") * _NC + lax.axis_index("c")
        base = wid * b_per_w

        pltpu.async_copy(idx_hbm.at[wid], idx_v, isem).wait()

        hi_mask = jnp.full((16,), -65536, jnp.int32)

        def widen(b):
            def row_body(rb, carry):
                for rr in range(8):
                    r = rb * 8 + rr
                    pk_row = pk_v.at[b, r]
                    out_row = rows_v.at[b, r]
                    for g in range(ngrp):
                        x = pk_row[pl.ds(16 * g, 16)]
                        out_row[pl.ds(32 * g, 16)] = lax.shift_left(x, 16)
                        out_row[pl.ds(32 * g + 16, 16)] = lax.bitwise_and(x, hi_mask)
                return carry

            lax.fori_loop(0, _CH // 8, row_body, 0)

        for b in range(_NBUF):
            pltpu.async_copy(tab_hbm.at[idx_v.at[b]], pk_v.at[b], gsem[b])

        for b in range(_NBUF):
            pltpu.make_async_copy(
                tab_hbm.at[idx_v.at[b]], pk_v.at[b], gsem[b]
            ).wait()
            widen(b)
            pltpu.async_copy(
                rows_v.at[b], out_hbm.at[pl.ds(base + b * _CH, _CH)], ssem[b]
            )
            pltpu.async_copy(
                tab_hbm.at[idx_v.at[b + _NBUF]], pk_v.at[b], gsem[b]
            )

        def outer(jj, carry):
            for b in range(_NBUF):
                j = jj * _NBUF + b
                pltpu.make_async_copy(
                    tab_hbm.at[idx_v.at[j]], pk_v.at[b], gsem[b]
                ).wait()
                pltpu.make_async_copy(
                    rows_v.at[b],
                    out_hbm.at[pl.ds(base + (j - _NBUF) * _CH, _CH)],
                    ssem[b],
                ).wait()
                widen(b)
                pltpu.async_copy(
                    rows_v.at[b], out_hbm.at[pl.ds(base + j * _CH, _CH)], ssem[b]
                )
                pltpu.async_copy(
                    tab_hbm.at[idx_v.at[j + _NBUF]], pk_v.at[b], gsem[b]
                )
            return carry

        lax.fori_loop(1, n_outer - 1, outer, 0)

        for b in range(_NBUF):
            j = (n_outer - 1) * _NBUF + b
            pltpu.make_async_copy(
                tab_hbm.at[idx_v.at[j]], pk_v.at[b], gsem[b]
            ).wait()
            pltpu.make_async_copy(
                rows_v.at[b],
                out_hbm.at[pl.ds(base + (j - _NBUF) * _CH, _CH)],
                ssem[b],
            ).wait()
            widen(b)
            pltpu.async_copy(
                rows_v.at[b], out_hbm.at[pl.ds(base + j * _CH, _CH)], ssem[b]
            )
        for b in range(_NBUF):
            j = (n_outer - 1) * _NBUF + b
            pltpu.make_async_copy(
                rows_v.at[b], out_hbm.at[pl.ds(base + j * _CH, _CH)], ssem[b]
            ).wait()

    return gather_kernel


def kernel(input, table):
    B, H = input.shape
    V, D = table.shape
    N = B * H
    b_per_w = N // _NW
    tp = table[:, _col_perm(D)].astype(jnp.bfloat16)
    t32 = jax.lax.bitcast_convert_type(tp.reshape(V, D // 2, 2), jnp.int32)
    idx = input.reshape(_NW, b_per_w // _CH, _CH).astype(jnp.int32)
    out = _make_gather(N, D)(t32, idx)
    return jax.lax.bitcast_convert_type(out, jnp.float32).reshape(B, H, D)


# speedup vs baseline: 3.9868x; 3.9766x over previous
import functools

import jax
import jax.numpy as jnp
from jax import lax
from jax.experimental import pallas as pl
from jax.experimental.pallas import tpu as pltpu
from jax.experimental.pallas import tpu_sc as plsc

_NC = 2
_NS = 16
_NW = _NC * _NS
_CH = 128
_NBUF = 5


@functools.lru_cache(maxsize=None)
def _make_gather(N, D):
    b_per_w = N // _NW
    n_ch = b_per_w // _CH
    n_outer = n_ch // _NBUF
    assert N == _NW * n_outer * _NBUF * _CH

    mesh = plsc.VectorSubcoreMesh(core_axis_name="c", subcore_axis_name="s")

    @functools.partial(
        pl.kernel,
        out_type=jax.ShapeDtypeStruct((N, D), jnp.float32),
        mesh=mesh,
        scratch_types=[
            pltpu.VMEM((n_ch, _CH), jnp.int32),
            pltpu.VMEM((_NBUF, _CH, D), jnp.float32),
            pltpu.SemaphoreType.DMA,
        ]
        + [pltpu.SemaphoreType.DMA for _ in range(_NBUF)]
        + [pltpu.SemaphoreType.DMA for _ in range(_NBUF)],
    )
    def gather_kernel(table_hbm, idx_hbm, out_hbm, idx_v, rows_v, isem, *sems):
        gsem = sems[:_NBUF]
        ssem = sems[_NBUF:]
        wid = lax.axis_index("s") * _NC + lax.axis_index("c")
        base = wid * b_per_w

        pltpu.async_copy(idx_hbm.at[wid], idx_v, isem).wait()

        for b in range(_NBUF):
            pltpu.async_copy(table_hbm.at[idx_v.at[b]], rows_v.at[b], gsem[b])

        def outer(jj, carry):
            for b in range(_NBUF):
                j = jj * _NBUF + b
                pltpu.make_async_copy(
                    table_hbm.at[idx_v.at[j]], rows_v.at[b], gsem[b]
                ).wait()
                pltpu.async_copy(
                    rows_v.at[b], out_hbm.at[pl.ds(base + j * _CH, _CH)], ssem[b]
                )
                pltpu.make_async_copy(
                    rows_v.at[b], out_hbm.at[pl.ds(base + j * _CH, _CH)], ssem[b]
                ).wait()
                pltpu.async_copy(
                    table_hbm.at[idx_v.at[j + _NBUF]], rows_v.at[b], gsem[b]
                )
            return carry

        lax.fori_loop(0, n_outer - 1, outer, 0)

        for b in range(_NBUF):
            j = (n_outer - 1) * _NBUF + b
            pltpu.make_async_copy(
                table_hbm.at[idx_v.at[j]], rows_v.at[b], gsem[b]
            ).wait()
            pltpu.async_copy(
                rows_v.at[b], out_hbm.at[pl.ds(base + j * _CH, _CH)], ssem[b]
            )
        for b in range(_NBUF):
            j = (n_outer - 1) * _NBUF + b
            pltpu.make_async_copy(
                rows_v.at[b], out_hbm.at[pl.ds(base + j * _CH, _CH)], ssem[b]
            ).wait()

    return gather_kernel


def kernel(input, table):
    B, H = input.shape
    V, D = table.shape
    N = B * H
    b_per_w = N // _NW
    idx = input.reshape(_NW, b_per_w // _CH, _CH).astype(jnp.int32)
    out = _make_gather(N, D)(table, idx)
    return out.reshape(B, H, D)
